# baseline (speedup 1.0000x reference)
import functools

import jax
import jax.numpy as jnp
import numpy as np
from jax import lax
from jax.experimental import pallas as pl
from jax.experimental.pallas import tpu as pltpu
from jax.experimental.pallas import tpu_sc as plsc

PC0 = np.float32(-75.2)
PS = np.float32(0.4)
FAR = np.float32(1e9)
POOL_EXTRA = np.float32(1.0)
K = 256
MAXB = 16
NC = 2
NS = 16
NW = NC * NS
L = 16


def _mesh():
    return plsc.VectorSubcoreMesh(
        core_axis_name="c", subcore_axis_name="s", num_cores=NC,
        num_subcores=NS)


@functools.lru_cache(maxsize=None)
def _build(n_pad, nb, groups, mboxes, nbatch):
    chunk = groups * L
    cpad = chunk + L
    qcap = groups * mboxes

    def stage1(xs_h, ys_h, zs_h, pb_h, par_h, bufs_h, cnts_h, pidx_h, pnum_h,
               st_v, cx_v, cy_v, cz_v, px_v, py_v, ci_v, q_v,
               par_v, buf_v, cnt_v, cnt_s, par_s, call_v, mrg_v, out_v,
               off_v, pn_v, sem):
        cid = lax.axis_index("c")
        sid = lax.axis_index("s")
        wid = cid * NS + sid
        base = wid * chunk
        pltpu.sync_copy(xs_h.at[pl.ds(base, chunk)], st_v.at[pl.ds(0, chunk)])
        pltpu.sync_copy(ys_h.at[pl.ds(base, chunk)],
                        st_v.at[pl.ds(chunk, chunk)])
        pltpu.sync_copy(zs_h.at[pl.ds(base, chunk)],
                        st_v.at[pl.ds(2 * chunk, chunk)])
        pltpu.sync_copy(pb_h.at[pl.ds(base, chunk)],
                        st_v.at[pl.ds(3 * chunk, chunk)])
        pltpu.sync_copy(par_h, par_v)

        def zero_body(i, carry):
            cnt_s[i] = 0
            return carry
        lax.fori_loop(0, nb, zero_body, 0)

        def par_body(i, carry):
            v = par_v[pl.ds(i * L, L)]
            for l in range(L):
                par_s[i * L + l] = v[l]
            return carry
        lax.fori_loop(0, 9 * nb // L, par_body, 0)

        iota = lax.iota(jnp.int32, L)
        lane0 = iota == 0

        def compact_body(g, cnts_c):
            gb = g * L
            xv = st_v[pl.ds(gb, L)]
            yv = st_v[pl.ds(chunk + gb, L)]
            zv = st_v[pl.ds(2 * chunk + gb, L)]
            bv = st_v[pl.ds(3 * chunk + gb, L)].astype(jnp.int32)
            pidx = base + gb + iota
            pxf = ((xv - PC0) / PS).astype(jnp.int32).astype(jnp.float32)
            pyf = ((yv - PC0) / PS).astype(jnp.int32).astype(jnp.float32)
            pcx = PC0 + (pxf + np.float32(0.5)) * PS
            pcy = PC0 + (pyf + np.float32(0.5)) * PS
            new = []
            for b in range(nbatch):
                cb = cnts_c[b]
                mb = bv == b
                nvalid = plsc.all_reduce_population_count(mb)[0]
                o = b * cpad
                plsc.store_compressed(cx_v.at[pl.ds(o + cb, L)], xv, mask=mb)
                plsc.store_compressed(cy_v.at[pl.ds(o + cb, L)], yv, mask=mb)
                plsc.store_compressed(cz_v.at[pl.ds(o + cb, L)], zv, mask=mb)
                plsc.store_compressed(px_v.at[pl.ds(o + cb, L)], pcx, mask=mb)
                plsc.store_compressed(py_v.at[pl.ds(o + cb, L)], pcy, mask=mb)
                plsc.store_compressed(ci_v.at[pl.ds(o + cb, L)], pidx,
                                      mask=mb)
                new.append(cb + nvalid)
            return tuple(new)
        cnts_c = lax.fori_loop(0, groups, compact_body,
                               tuple(jnp.int32(0) for _ in range(nbatch)))

        far = jnp.full((L,), FAR, jnp.float32)
        for b in range(nbatch):
            px_v[pl.ds(b * cpad + cnts_c[b], L)] = far
            py_v[pl.ds(b * cpad + cnts_c[b], L)] = far

        qns = []
        for b in range(nbatch):
            gcount = (cnts_c[b] + (L - 1)) // L
            o = b * cpad
            qbase = b * qcap

            def scan_g(gq, qn, b=b, o=o, qbase=qbase):
                pcx = px_v[pl.ds(o + gq * L, L)]
                pcy = py_v[pl.ds(o + gq * L, L)]
                gm = gq * mboxes

                def scan_blk(blk, qn):
                    s0 = (b * mboxes + blk * L) * 9
                    bits = jnp.int32(0)
                    for jj in range(L):
                        mb9 = s0 + jj * 9
                        bx = jnp.full((L,), par_s[mb9])
                        by = jnp.full((L,), par_s[mb9 + 1])
                        rc = jnp.full((L,), par_s[mb9 + 8])
                        cov = ((jnp.abs(pcx - bx) <= rc)
                               & (jnp.abs(pcy - by) <= rc))
                        pc = plsc.all_reduce_population_count(cov)[0]
                        bits = bits | ((pc > 0).astype(jnp.int32) << jj)
                    bvec = ((jnp.full((L,), bits) >> iota) & 1) != 0
                    nh = plsc.all_reduce_population_count(bvec)[0]
                    plsc.store_compressed(
                        q_v.at[pl.ds(qbase + qn, L)],
                        gm + blk * L + iota, mask=bvec)
                    return qn + nh
                return lax.fori_loop(0, mboxes // L, scan_blk, qn)
            qns.append(lax.fori_loop(0, gcount, scan_g, jnp.int32(0)))

        for b in range(nbatch):
            o = b * cpad

            def drain(k, carry, b=b, o=o):
                prev_g, rank = carry
                packed = plsc.load_gather(
                    q_v, [jnp.full((L,), b * qcap + k, jnp.int32)])[0]
                gq = packed // mboxes
                mm = packed - gq * mboxes
                m = b * mboxes + mm
                mb9 = m * 9
                rank = jnp.where(gq != prev_g, jnp.zeros((L,), jnp.int32),
                                 rank)
                gb = o + gq * L
                pcx = px_v[pl.ds(gb, L)]
                pcy = py_v[pl.ds(gb, L)]
                bx = jnp.full((L,), par_s[mb9])
                by = jnp.full((L,), par_s[mb9 + 1])
                rc = jnp.full((L,), par_s[mb9 + 8])
                cov = (jnp.abs(pcx - bx) <= rc) & (jnp.abs(pcy - by) <= rc)
                cand = cov & (rank < MAXB)
                xv = cx_v[pl.ds(gb, L)]
                yv = cy_v[pl.ds(gb, L)]
                zv = cz_v[pl.ds(gb, L)]
                pidx = ci_v[pl.ds(gb, L)]
                bz = jnp.full((L,), par_s[mb9 + 2])
                hdx = jnp.full((L,), par_s[mb9 + 3])
                hdy = jnp.full((L,), par_s[mb9 + 4])
                hdz = jnp.full((L,), par_s[mb9 + 5])
                cs = jnp.full((L,), par_s[mb9 + 6])
                sn = jnp.full((L,), par_s[mb9 + 7])
                dx = xv - bx
                dy = yv - by
                dz = zv - bz
                lx = dx * cs + dy * sn
                ly = dy * cs - dx * sn
                msk = (cand & (jnp.abs(lx) <= hdx)
                       & (jnp.abs(ly) <= hdy) & (jnp.abs(dz) <= hdz))
                mi = msk.astype(jnp.int32)
                c0 = cnt_s[m]
                pos = plsc.cumsum(mi)
                wpos = c0 + pos - 1
                wmask = msk & (wpos < K)
                wpos = jnp.where(wmask, wpos, 0)
                plsc.store_scatter(buf_v, [m * K + wpos], pidx, mask=wmask)
                cnt_s[m] = c0 + pos[L - 1]
                return gq, rank + cov.astype(jnp.int32)
            lax.fori_loop(0, qns[b], drain,
                          (jnp.int32(-1), jnp.zeros((L,), jnp.int32)))

        def mirror_body(i, carry):
            c = cnt_s[i]
            plsc.store_scatter(
                cnt_v, [jnp.full((L,), i, jnp.int32)],
                jnp.full((L,), c, jnp.int32), mask=iota == 0)
            return carry
        lax.fori_loop(0, nb, mirror_body, 0)
        pltpu.sync_copy(buf_v, bufs_h.at[pl.ds(wid * nb * K, nb * K)])
        pltpu.sync_copy(cnt_v, cnts_h.at[pl.ds(wid * nb, nb)])

        plsc.subcore_barrier()
        hbase = cid * NS * nb * K
        pltpu.sync_copy(cnts_h, call_v)

        def merge_body(i, carry):
            m = sid * (nb // NS) + i
            copies = [
                pltpu.async_copy(
                    bufs_h.at[pl.ds(hbase + w * nb * K + m * K, K)],
                    mrg_v.at[pl.ds(w * K, K)], sem)
                for w in range(NS)
            ]
            mv = jnp.full((L,), m, jnp.int32)
            cw = jnp.minimum(
                plsc.load_gather(call_v, [(cid * NS + iota) * nb + mv]), K)
            s = plsc.cumsum(cw)
            total = s[L - 1]
            off_v[pl.ds(0, L)] = s - cw
            pnum = jnp.minimum(total, K)
            for cp in copies:
                cp.wait()
            for jg in range(K // L):
                j = jg * L + iota
                w = jnp.zeros((L,), jnp.int32)
                for st in (8, 4, 2, 1):
                    nxt = w + st
                    ow = plsc.load_gather(off_v, [nxt])
                    w = jnp.where(ow <= j, nxt, w)
                og = plsc.load_gather(off_v, [w])
                rel = jnp.minimum(jnp.maximum(j - og, 0), K - 1)
                src = plsc.load_gather(mrg_v, [w * K + rel])
                out_v[pl.ds(jg * L, L)] = jnp.where(j < pnum, src, 0)
            pltpu.sync_copy(out_v, pidx_h.at[pl.ds((cid * nb + m) * K, K)])
            plsc.store_scatter(
                pn_v, [jnp.full((L,), i, jnp.int32)],
                jnp.full((L,), pnum, jnp.int32), mask=iota == 0)
            return carry

        lax.fori_loop(0, nb // NS, merge_body, 0)
        pltpu.sync_copy(pn_v.at[pl.ds(0, 8)],
                        pnum_h.at[pl.ds(cid * nb + sid * (nb // NS), 8)])

    s1 = pl.kernel(
        stage1,
        out_type=[jax.ShapeDtypeStruct((NW * nb * K,), jnp.int32),
                  jax.ShapeDtypeStruct((NW * nb,), jnp.int32),
                  jax.ShapeDtypeStruct((NC * nb * K,), jnp.int32),
                  jax.ShapeDtypeStruct((NC * nb,), jnp.int32)],
        mesh=_mesh(),
        compiler_params=pltpu.CompilerParams(needs_layout_passes=False),
        scratch_types=[
            pltpu.VMEM((4 * chunk,), jnp.float32),
            pltpu.VMEM((nbatch * cpad,), jnp.float32),
            pltpu.VMEM((nbatch * cpad,), jnp.float32),
            pltpu.VMEM((nbatch * cpad,), jnp.float32),
            pltpu.VMEM((nbatch * cpad,), jnp.float32),
            pltpu.VMEM((nbatch * cpad,), jnp.float32),
            pltpu.VMEM((nbatch * cpad,), jnp.int32),
            pltpu.VMEM((nbatch * qcap + L,), jnp.int32),
            pltpu.VMEM((9 * nb,), jnp.float32),
            pltpu.VMEM((nb * K,), jnp.int32),
            pltpu.VMEM((nb,), jnp.int32),
            pltpu.SMEM((nb,), jnp.int32),
            pltpu.SMEM((9 * nb,), jnp.float32),
            pltpu.VMEM((NW * nb,), jnp.int32),
            pltpu.VMEM((NS * K,), jnp.int32),
            pltpu.VMEM((K,), jnp.int32),
            pltpu.VMEM((L,), jnp.int32),
            pltpu.VMEM((L,), jnp.int32),
            pltpu.SemaphoreType.DMA,
        ],
    )
    return s1


def kernel(points, boxes3d):
    n = points.shape[0]
    b, m = boxes3d.shape[0], boxes3d.shape[1]
    nb = b * m
    groups = -(-n // (NW * L))
    n_pad = NW * groups * L
    s1 = _build(n_pad, nb, groups, m, b)

    ctr = boxes3d[..., 0:3]
    edx = boxes3d[..., 4] + 2.0 * POOL_EXTRA
    edy = boxes3d[..., 3] + 2.0 * POOL_EXTRA
    edz = boxes3d[..., 5] + 2.0 * POOL_EXTRA
    ry = -boxes3d[..., 6] - np.float32(np.pi / 2.0)
    radius = 0.5 * jnp.sqrt(edx ** 2 + edy ** 2)
    par = jnp.stack([
        ctr[..., 0].reshape(nb), ctr[..., 1].reshape(nb),
        ctr[..., 2].reshape(nb),
        (edx * 0.5).reshape(nb), (edy * 0.5).reshape(nb),
        (edz * 0.5).reshape(nb),
        jnp.cos(ry).reshape(nb), jnp.sin(ry).reshape(nb),
        (radius + PS).reshape(nb),
    ], axis=1).reshape(9 * nb).astype(jnp.float32)

    pad = n_pad - n
    xs = jnp.concatenate([points[:, 1], jnp.zeros((pad,), jnp.float32)])
    ys = jnp.concatenate([points[:, 2], jnp.zeros((pad,), jnp.float32)])
    zs = jnp.concatenate([points[:, 3], jnp.zeros((pad,), jnp.float32)])
    pb = jnp.concatenate([points[:, 0], jnp.full((pad,), b, jnp.float32)])

    _, _, pidx2, pnum2 = s1(xs, ys, zs, pb, par)
    half = pidx2.reshape(NC, nb, K)
    c0 = pnum2[:nb]
    c1 = pnum2[nb:]
    j = jnp.arange(K, dtype=jnp.int32)[None, :]
    rel_b = jnp.clip(j - c0[:, None], 0, K - 1)
    from_b = jnp.take_along_axis(half[1], rel_b, axis=1)
    total = jnp.minimum(c0 + c1, K)
    out = jnp.where(j < c0[:, None], half[0], from_b)
    out = jnp.where(j < total[:, None], out, 0)
    return out.reshape(b, m, K), total.reshape(b, m)


# speedup vs baseline: 1.1706x; 1.1706x over previous
import functools

import jax
import jax.numpy as jnp
import numpy as np
from jax import lax
from jax.experimental import pallas as pl
from jax.experimental.pallas import tpu as pltpu
from jax.experimental.pallas import tpu_sc as plsc

PC0 = np.float32(-75.2)
PS = np.float32(0.4)
FAR = np.float32(1e9)
POOL_EXTRA = np.float32(1.0)
K = 256
MAXB = 16
NC = 2
NS = 16
NW = NC * NS
L = 16


def _mesh():
    return plsc.VectorSubcoreMesh(
        core_axis_name="c", subcore_axis_name="s", num_cores=NC,
        num_subcores=NS)


@functools.lru_cache(maxsize=None)
def _build(n_pad, nb, groups, mboxes, nbatch):
    chunk = groups * L
    cpad = chunk + L
    qcap = groups * mboxes

    def stage1(xs_h, ys_h, zs_h, pb_h, par_h, bufs_h, cnts_h,
               st_v, cx_v, cy_v, cz_v, px_v, py_v, ci_v, q_v,
               par_v, buf_v, cnt_v, cnt_s, par_s):
        wid = lax.axis_index("s") * NC + lax.axis_index("c")
        base = wid * chunk
        pltpu.sync_copy(xs_h.at[pl.ds(base, chunk)], st_v.at[pl.ds(0, chunk)])
        pltpu.sync_copy(ys_h.at[pl.ds(base, chunk)],
                        st_v.at[pl.ds(chunk, chunk)])
        pltpu.sync_copy(zs_h.at[pl.ds(base, chunk)],
                        st_v.at[pl.ds(2 * chunk, chunk)])
        pltpu.sync_copy(pb_h.at[pl.ds(base, chunk)],
                        st_v.at[pl.ds(3 * chunk, chunk)])
        pltpu.sync_copy(par_h, par_v)

        def zero_body(i, carry):
            cnt_s[i] = 0
            return carry
        lax.fori_loop(0, nb, zero_body, 0)

        def par_body(i, carry):
            v = par_v[pl.ds(i * L, L)]
            for l in range(L):
                par_s[i * L + l] = v[l]
            return carry
        lax.fori_loop(0, 9 * nb // L, par_body, 0)

        iota = lax.iota(jnp.int32, L)
        lane0 = iota == 0

        def compact_body(g, cnts_c):
            gb = g * L
            xv = st_v[pl.ds(gb, L)]
            yv = st_v[pl.ds(chunk + gb, L)]
            zv = st_v[pl.ds(2 * chunk + gb, L)]
            bv = st_v[pl.ds(3 * chunk + gb, L)].astype(jnp.int32)
            pidx = base + gb + iota
            pxf = ((xv - PC0) / PS).astype(jnp.int32).astype(jnp.float32)
            pyf = ((yv - PC0) / PS).astype(jnp.int32).astype(jnp.float32)
            pcx = PC0 + (pxf + np.float32(0.5)) * PS
            pcy = PC0 + (pyf + np.float32(0.5)) * PS
            new = []
            for b in range(nbatch):
                cb = cnts_c[b]
                mb = bv == b
                nvalid = plsc.all_reduce_population_count(mb)[0]
                o = b * cpad
                plsc.store_compressed(cx_v.at[pl.ds(o + cb, L)], xv, mask=mb)
                plsc.store_compressed(cy_v.at[pl.ds(o + cb, L)], yv, mask=mb)
                plsc.store_compressed(cz_v.at[pl.ds(o + cb, L)], zv, mask=mb)
                plsc.store_compressed(px_v.at[pl.ds(o + cb, L)], pcx, mask=mb)
                plsc.store_compressed(py_v.at[pl.ds(o + cb, L)], pcy, mask=mb)
                plsc.store_compressed(ci_v.at[pl.ds(o + cb, L)], pidx,
                                      mask=mb)
                new.append(cb + nvalid)
            return tuple(new)
        cnts_c = lax.fori_loop(0, groups, compact_body,
                               tuple(jnp.int32(0) for _ in range(nbatch)))

        far = jnp.full((L,), FAR, jnp.float32)
        for b in range(nbatch):
            px_v[pl.ds(b * cpad + cnts_c[b], L)] = far
            py_v[pl.ds(b * cpad + cnts_c[b], L)] = far

        qns = []
        for b in range(nbatch):
            gcount = (cnts_c[b] + (L - 1)) // L
            o = b * cpad
            qbase = b * qcap

            def scan_g(gq, qn, b=b, o=o, qbase=qbase):
                pcx = px_v[pl.ds(o + gq * L, L)]
                pcy = py_v[pl.ds(o + gq * L, L)]
                gm = gq * mboxes

                def scan_blk(blk, qn):
                    s0 = (b * mboxes + blk * L) * 9
                    bits = jnp.int32(0)
                    for jj in range(L):
                        mb9 = s0 + jj * 9
                        bx = jnp.full((L,), par_s[mb9])
                        by = jnp.full((L,), par_s[mb9 + 1])
                        rc = jnp.full((L,), par_s[mb9 + 8])
                        cov = ((jnp.abs(pcx - bx) <= rc)
                               & (jnp.abs(pcy - by) <= rc))
                        pc = plsc.all_reduce_population_count(cov)[0]
                        bits = bits | ((pc > 0).astype(jnp.int32) << jj)
                    bvec = ((jnp.full((L,), bits) >> iota) & 1) != 0
                    nh = plsc.all_reduce_population_count(bvec)[0]
                    plsc.store_compressed(
                        q_v.at[pl.ds(qbase + qn, L)],
                        gm + blk * L + iota, mask=bvec)
                    return qn + nh
                return lax.fori_loop(0, mboxes // L, scan_blk, qn)
            qns.append(lax.fori_loop(0, gcount, scan_g, jnp.int32(0)))

        for b in range(nbatch):
            o = b * cpad

            def drain(k, carry, b=b, o=o):
                prev_g, rank, packed = carry
                nextp = plsc.load_gather(
                    q_v, [jnp.full((L,), b * qcap + k + 1, jnp.int32)])[0]
                gq = packed // mboxes
                mm = packed - gq * mboxes
                m = b * mboxes + mm
                mb9 = m * 9
                rank = jnp.where(gq != prev_g, jnp.zeros((L,), jnp.int32),
                                 rank)
                gb = o + gq * L
                pcx = px_v[pl.ds(gb, L)]
                pcy = py_v[pl.ds(gb, L)]
                bx = jnp.full((L,), par_s[mb9])
                by = jnp.full((L,), par_s[mb9 + 1])
                rc = jnp.full((L,), par_s[mb9 + 8])
                cov = (jnp.abs(pcx - bx) <= rc) & (jnp.abs(pcy - by) <= rc)
                cand = cov & (rank < MAXB)
                xv = cx_v[pl.ds(gb, L)]
                yv = cy_v[pl.ds(gb, L)]
                zv = cz_v[pl.ds(gb, L)]
                pidx = ci_v[pl.ds(gb, L)]
                bz = jnp.full((L,), par_s[mb9 + 2])
                hdx = jnp.full((L,), par_s[mb9 + 3])
                hdy = jnp.full((L,), par_s[mb9 + 4])
                hdz = jnp.full((L,), par_s[mb9 + 5])
                cs = jnp.full((L,), par_s[mb9 + 6])
                sn = jnp.full((L,), par_s[mb9 + 7])
                dx = xv - bx
                dy = yv - by
                dz = zv - bz
                lx = dx * cs + dy * sn
                ly = dy * cs - dx * sn
                msk = (cand & (jnp.abs(lx) <= hdx)
                       & (jnp.abs(ly) <= hdy) & (jnp.abs(dz) <= hdz))
                mi = msk.astype(jnp.int32)
                c0 = cnt_s[m]
                pos = plsc.cumsum(mi)
                wpos = c0 + pos - 1
                wmask = msk & (wpos < K)
                wpos = jnp.where(wmask, wpos, 0)
                plsc.store_scatter(buf_v, [m * K + wpos], pidx, mask=wmask)
                cnt_s[m] = c0 + pos[L - 1]
                return gq, rank + cov.astype(jnp.int32), nextp
            first = plsc.load_gather(
                q_v, [jnp.full((L,), b * qcap, jnp.int32)])[0]
            lax.fori_loop(0, qns[b], drain,
                          (jnp.int32(-1), jnp.zeros((L,), jnp.int32), first))

        def mirror_body(i, carry):
            c = cnt_s[i]
            plsc.store_scatter(
                cnt_v, [jnp.full((L,), i, jnp.int32)],
                jnp.full((L,), c, jnp.int32), mask=iota == 0)
            return carry
        lax.fori_loop(0, nb, mirror_body, 0)
        pltpu.sync_copy(buf_v, bufs_h.at[pl.ds(wid * nb * K, nb * K)])
        pltpu.sync_copy(cnt_v, cnts_h.at[pl.ds(wid * nb, nb)])

    s1 = pl.kernel(
        stage1,
        out_type=[jax.ShapeDtypeStruct((NW * nb * K,), jnp.int32),
                  jax.ShapeDtypeStruct((NW * nb,), jnp.int32)],
        mesh=_mesh(),
        compiler_params=pltpu.CompilerParams(needs_layout_passes=False),
        scratch_types=[
            pltpu.VMEM((4 * chunk,), jnp.float32),
            pltpu.VMEM((nbatch * cpad,), jnp.float32),
            pltpu.VMEM((nbatch * cpad,), jnp.float32),
            pltpu.VMEM((nbatch * cpad,), jnp.float32),
            pltpu.VMEM((nbatch * cpad,), jnp.float32),
            pltpu.VMEM((nbatch * cpad,), jnp.float32),
            pltpu.VMEM((nbatch * cpad,), jnp.int32),
            pltpu.VMEM((nbatch * qcap + L,), jnp.int32),
            pltpu.VMEM((9 * nb,), jnp.float32),
            pltpu.VMEM((nb * K,), jnp.int32),
            pltpu.VMEM((nb,), jnp.int32),
            pltpu.SMEM((nb,), jnp.int32),
            pltpu.SMEM((9 * nb,), jnp.float32),
        ],
    )

    boxes_per_w = nb // NW

    def stage2(bufs_h, cnts_h, pidx_h, pnum_h,
               cnts_v, vbuf, out_v, off_v, cw_v, pn_v, sem):
        wid = lax.axis_index("s") * NC + lax.axis_index("c")
        pltpu.sync_copy(cnts_h, cnts_v)
        iota = lax.iota(jnp.int32, L)

        def box_body(i, carry):
            m = wid * boxes_per_w + i
            copies = [
                pltpu.async_copy(
                    bufs_h.at[pl.ds(w * nb * K + m * K, K)],
                    vbuf.at[pl.ds(w * K, K)], sem)
                for w in range(NW)
            ]
            mv = jnp.full((L,), m, jnp.int32)
            c_lo = jnp.minimum(plsc.load_gather(cnts_v, [iota * nb + mv]), K)
            c_hi = jnp.minimum(
                plsc.load_gather(cnts_v, [(iota + L) * nb + mv]), K)
            t_lo = jnp.sum(c_lo)
            s_lo = plsc.cumsum(c_lo)
            s_hi = plsc.cumsum(c_hi) + t_lo
            total = jnp.sum(c_hi) + t_lo
            off_v[pl.ds(0, L)] = s_lo - c_lo
            off_v[pl.ds(L, L)] = s_hi - c_hi
            cw_v[pl.ds(0, L)] = c_lo
            cw_v[pl.ds(L, L)] = c_hi
            pnum = jnp.minimum(total, K)
            for cp in copies:
                cp.wait()
            for jg in range(K // L):
                j = jg * L + iota
                w = jnp.zeros((L,), jnp.int32)
                for s in (16, 8, 4, 2, 1):
                    nxt = w + s
                    ow = plsc.load_gather(off_v, [nxt])
                    w = jnp.where(ow <= j, nxt, w)
                og = plsc.load_gather(off_v, [w])
                rel = jnp.minimum(jnp.maximum(j - og, 0), K - 1)
                src = plsc.load_gather(vbuf, [w * K + rel])
                out_v[pl.ds(jg * L, L)] = jnp.where(j < pnum, src, 0)
            pltpu.sync_copy(out_v, pidx_h.at[pl.ds(m * K, K)])
            plsc.store_scatter(
                pn_v, [jnp.full((L,), i, jnp.int32)],
                jnp.full((L,), pnum, jnp.int32), mask=iota == 0)
            return carry
        lax.fori_loop(0, boxes_per_w, box_body, 0)
        pltpu.sync_copy(pn_v.at[pl.ds(0, 8)], pnum_h.at[pl.ds(wid * 8, 8)])

    s2 = pl.kernel(
        stage2,
        out_type=[jax.ShapeDtypeStruct((nb * K,), jnp.int32),
                  jax.ShapeDtypeStruct((NW * 8,), jnp.int32)],
        mesh=_mesh(),
        compiler_params=pltpu.CompilerParams(needs_layout_passes=False),
        scratch_types=[
            pltpu.VMEM((NW * nb,), jnp.int32),
            pltpu.VMEM((NW * K,), jnp.int32),
            pltpu.VMEM((K,), jnp.int32),
            pltpu.VMEM((2 * L,), jnp.int32),
            pltpu.VMEM((2 * L,), jnp.int32),
            pltpu.VMEM((L,), jnp.int32),
            pltpu.SemaphoreType.DMA,
        ],
    )
    return s1, s2


def kernel(points, boxes3d):
    n = points.shape[0]
    b, m = boxes3d.shape[0], boxes3d.shape[1]
    nb = b * m
    groups = -(-n // (NW * L))
    n_pad = NW * groups * L
    s1, s2 = _build(n_pad, nb, groups, m, b)

    ctr = boxes3d[..., 0:3]
    edx = boxes3d[..., 4] + 2.0 * POOL_EXTRA
    edy = boxes3d[..., 3] + 2.0 * POOL_EXTRA
    edz = boxes3d[..., 5] + 2.0 * POOL_EXTRA
    ry = -boxes3d[..., 6] - np.float32(np.pi / 2.0)
    radius = 0.5 * jnp.sqrt(edx ** 2 + edy ** 2)
    par = jnp.stack([
        ctr[..., 0].reshape(nb), ctr[..., 1].reshape(nb),
        ctr[..., 2].reshape(nb),
        (edx * 0.5).reshape(nb), (edy * 0.5).reshape(nb),
        (edz * 0.5).reshape(nb),
        jnp.cos(ry).reshape(nb), jnp.sin(ry).reshape(nb),
        (radius + PS).reshape(nb),
    ], axis=1).reshape(9 * nb).astype(jnp.float32)

    pad = n_pad - n
    xs = jnp.concatenate([points[:, 1], jnp.zeros((pad,), jnp.float32)])
    ys = jnp.concatenate([points[:, 2], jnp.zeros((pad,), jnp.float32)])
    zs = jnp.concatenate([points[:, 3], jnp.zeros((pad,), jnp.float32)])
    pb = jnp.concatenate([points[:, 0], jnp.full((pad,), b, jnp.float32)])

    bufs, cnts = s1(xs, ys, zs, pb, par)
    pidx, pnum = s2(bufs, cnts)
    boxes_per_w = nb // NW
    pooled_idx = pidx.reshape(b, m, K)
    pooled_num = pnum.reshape(NW, 8)[:, :boxes_per_w].reshape(b, m)
    return pooled_idx, pooled_num


# speedup vs baseline: 1.1959x; 1.0216x over previous
import functools

import jax
import jax.numpy as jnp
import numpy as np
from jax import lax
from jax.experimental import pallas as pl
from jax.experimental.pallas import tpu as pltpu
from jax.experimental.pallas import tpu_sc as plsc

PC0 = np.float32(-75.2)
PS = np.float32(0.4)
FAR = np.float32(1e9)
POOL_EXTRA = np.float32(1.0)
K = 256
MAXB = 16
NC = 2
NS = 16
NW = NC * NS
L = 16


def _mesh():
    return plsc.VectorSubcoreMesh(
        core_axis_name="c", subcore_axis_name="s", num_cores=NC,
        num_subcores=NS)


@functools.lru_cache(maxsize=None)
def _build(n_pad, nb, groups, mboxes, nbatch):
    chunk = groups * L
    cpad = chunk + L
    qcap = groups * mboxes

    def stage1(xs_h, ys_h, zs_h, pb_h, par_h, bufs_h, cnts_h,
               st_v, cx_v, cy_v, cz_v, px_v, py_v, ci_v, q_v,
               par_v, buf_v, cnt_v, cnt_s, par_s):
        wid = lax.axis_index("s") * NC + lax.axis_index("c")
        base = wid * chunk
        pltpu.sync_copy(xs_h.at[pl.ds(base, chunk)], st_v.at[pl.ds(0, chunk)])
        pltpu.sync_copy(ys_h.at[pl.ds(base, chunk)],
                        st_v.at[pl.ds(chunk, chunk)])
        pltpu.sync_copy(zs_h.at[pl.ds(base, chunk)],
                        st_v.at[pl.ds(2 * chunk, chunk)])
        pltpu.sync_copy(pb_h.at[pl.ds(base, chunk)],
                        st_v.at[pl.ds(3 * chunk, chunk)])
        pltpu.sync_copy(par_h, par_v)

        def zero_body(i, carry):
            cnt_s[i] = 0
            return carry
        lax.fori_loop(0, nb, zero_body, 0)

        def par_body(i, carry):
            v = par_v[pl.ds(i * L, L)]
            for l in range(L):
                par_s[i * L + l] = v[l]
            return carry
        lax.fori_loop(0, 9 * nb // L, par_body, 0)

        iota = lax.iota(jnp.int32, L)
        lane0 = iota == 0

        def compact_body(g, cnts_c):
            gb = g * L
            xv = st_v[pl.ds(gb, L)]
            yv = st_v[pl.ds(chunk + gb, L)]
            zv = st_v[pl.ds(2 * chunk + gb, L)]
            bv = st_v[pl.ds(3 * chunk + gb, L)].astype(jnp.int32)
            pidx = base + gb + iota
            pxf = ((xv - PC0) / PS).astype(jnp.int32).astype(jnp.float32)
            pyf = ((yv - PC0) / PS).astype(jnp.int32).astype(jnp.float32)
            pcx = PC0 + (pxf + np.float32(0.5)) * PS
            pcy = PC0 + (pyf + np.float32(0.5)) * PS
            new = []
            for b in range(nbatch):
                cb = cnts_c[b]
                mb = bv == b
                nvalid = plsc.all_reduce_population_count(mb)[0]
                o = b * cpad
                plsc.store_compressed(cx_v.at[pl.ds(o + cb, L)], xv, mask=mb)
                plsc.store_compressed(cy_v.at[pl.ds(o + cb, L)], yv, mask=mb)
                plsc.store_compressed(cz_v.at[pl.ds(o + cb, L)], zv, mask=mb)
                plsc.store_compressed(px_v.at[pl.ds(o + cb, L)], pcx, mask=mb)
                plsc.store_compressed(py_v.at[pl.ds(o + cb, L)], pcy, mask=mb)
                plsc.store_compressed(ci_v.at[pl.ds(o + cb, L)], pidx,
                                      mask=mb)
                new.append(cb + nvalid)
            return tuple(new)
        cnts_c = lax.fori_loop(0, groups, compact_body,
                               tuple(jnp.int32(0) for _ in range(nbatch)))

        far = jnp.full((L,), FAR, jnp.float32)
        for b in range(nbatch):
            px_v[pl.ds(b * cpad + cnts_c[b], L)] = far
            py_v[pl.ds(b * cpad + cnts_c[b], L)] = far

        qns = []
        for b in range(nbatch):
            gcount = (cnts_c[b] + (L - 1)) // L
            o = b * cpad
            qbase = b * qcap

            def scan_g(gq, qn, b=b, o=o, qbase=qbase):
                pcx = px_v[pl.ds(o + gq * L, L)]
                pcy = py_v[pl.ds(o + gq * L, L)]
                gm = gq * mboxes

                def scan_blk(blk, qn):
                    s0 = (b * mboxes + blk * L) * 9
                    bits = jnp.int32(0)
                    for jj in range(L):
                        mb9 = s0 + jj * 9
                        bx = jnp.full((L,), par_s[mb9])
                        by = jnp.full((L,), par_s[mb9 + 1])
                        rc = jnp.full((L,), par_s[mb9 + 8])
                        cov = jnp.maximum(jnp.abs(pcx - bx),
                                          jnp.abs(pcy - by)) <= rc
                        pc = plsc.all_reduce_population_count(cov)[0]
                        bits = bits | ((pc > 0).astype(jnp.int32) << jj)
                    bvec = ((jnp.full((L,), bits) >> iota) & 1) != 0
                    nh = plsc.all_reduce_population_count(bvec)[0]
                    plsc.store_compressed(
                        q_v.at[pl.ds(qbase + qn, L)],
                        gm + blk * L + iota, mask=bvec)
                    return qn + nh
                return lax.fori_loop(0, mboxes // L, scan_blk, qn)
            qns.append(lax.fori_loop(0, gcount, scan_g, jnp.int32(0)))

        for b in range(nbatch):
            o = b * cpad

            def drain(k, carry, b=b, o=o):
                prev_g, rank, packed = carry
                nextp = plsc.load_gather(
                    q_v, [jnp.full((L,), b * qcap + k + 1, jnp.int32)])[0]
                gq = packed // mboxes
                mm = packed - gq * mboxes
                m = b * mboxes + mm
                mb9 = m * 9
                rank = jnp.where(gq != prev_g, jnp.zeros((L,), jnp.int32),
                                 rank)
                gb = o + gq * L
                pcx = px_v[pl.ds(gb, L)]
                pcy = py_v[pl.ds(gb, L)]
                bx = jnp.full((L,), par_s[mb9])
                by = jnp.full((L,), par_s[mb9 + 1])
                rc = jnp.full((L,), par_s[mb9 + 8])
                cov = jnp.maximum(jnp.abs(pcx - bx),
                                  jnp.abs(pcy - by)) <= rc
                cand = cov & (rank < MAXB)
                xv = cx_v[pl.ds(gb, L)]
                yv = cy_v[pl.ds(gb, L)]
                zv = cz_v[pl.ds(gb, L)]
                pidx = ci_v[pl.ds(gb, L)]
                bz = jnp.full((L,), par_s[mb9 + 2])
                hdx = jnp.full((L,), par_s[mb9 + 3])
                hdy = jnp.full((L,), par_s[mb9 + 4])
                hdz = jnp.full((L,), par_s[mb9 + 5])
                cs = jnp.full((L,), par_s[mb9 + 6])
                sn = jnp.full((L,), par_s[mb9 + 7])
                dx = xv - bx
                dy = yv - by
                dz = zv - bz
                lx = dx * cs + dy * sn
                ly = dy * cs - dx * sn
                msk = (cand & (jnp.abs(lx) <= hdx)
                       & (jnp.abs(ly) <= hdy) & (jnp.abs(dz) <= hdz))
                mi = msk.astype(jnp.int32)
                c0 = cnt_s[m]
                pos = plsc.cumsum(mi)
                wpos = c0 + pos - 1
                wmask = msk & (wpos < K)
                wpos = jnp.where(wmask, wpos, 0)
                plsc.store_scatter(buf_v, [m * K + wpos], pidx, mask=wmask)
                cnt_s[m] = c0 + pos[L - 1]
                return gq, rank + cov.astype(jnp.int32), nextp
            first = plsc.load_gather(
                q_v, [jnp.full((L,), b * qcap, jnp.int32)])[0]
            lax.fori_loop(0, qns[b], drain,
                          (jnp.int32(-1), jnp.zeros((L,), jnp.int32), first))

        def mirror_body(i, carry):
            c = cnt_s[i]
            plsc.store_scatter(
                cnt_v, [jnp.full((L,), i, jnp.int32)],
                jnp.full((L,), c, jnp.int32), mask=iota == 0)
            return carry
        lax.fori_loop(0, nb, mirror_body, 0)
        pltpu.sync_copy(buf_v, bufs_h.at[pl.ds(wid * nb * K, nb * K)])
        pltpu.sync_copy(cnt_v, cnts_h.at[pl.ds(wid * nb, nb)])

    s1 = pl.kernel(
        stage1,
        out_type=[jax.ShapeDtypeStruct((NW * nb * K,), jnp.int32),
                  jax.ShapeDtypeStruct((NW * nb,), jnp.int32)],
        mesh=_mesh(),
        compiler_params=pltpu.CompilerParams(needs_layout_passes=False),
        scratch_types=[
            pltpu.VMEM((4 * chunk,), jnp.float32),
            pltpu.VMEM((nbatch * cpad,), jnp.float32),
            pltpu.VMEM((nbatch * cpad,), jnp.float32),
            pltpu.VMEM((nbatch * cpad,), jnp.float32),
            pltpu.VMEM((nbatch * cpad,), jnp.float32),
            pltpu.VMEM((nbatch * cpad,), jnp.float32),
            pltpu.VMEM((nbatch * cpad,), jnp.int32),
            pltpu.VMEM((nbatch * qcap + L,), jnp.int32),
            pltpu.VMEM((9 * nb,), jnp.float32),
            pltpu.VMEM((nb * K,), jnp.int32),
            pltpu.VMEM((nb,), jnp.int32),
            pltpu.SMEM((nb,), jnp.int32),
            pltpu.SMEM((9 * nb,), jnp.float32),
        ],
    )

    boxes_per_w = nb // NW

    def stage2(bufs_h, cnts_h, pidx_h, pnum_h,
               cnts_v, vbuf, out_v, off_v, cw_v, pn_v, sem):
        wid = lax.axis_index("s") * NC + lax.axis_index("c")
        pltpu.sync_copy(cnts_h, cnts_v)
        iota = lax.iota(jnp.int32, L)

        def box_body(i, carry):
            m = wid * boxes_per_w + i
            copies = [
                pltpu.async_copy(
                    bufs_h.at[pl.ds(w * nb * K + m * K, K)],
                    vbuf.at[pl.ds(w * K, K)], sem)
                for w in range(NW)
            ]
            mv = jnp.full((L,), m, jnp.int32)
            c_lo = jnp.minimum(plsc.load_gather(cnts_v, [iota * nb + mv]), K)
            c_hi = jnp.minimum(
                plsc.load_gather(cnts_v, [(iota + L) * nb + mv]), K)
            t_lo = jnp.sum(c_lo)
            s_lo = plsc.cumsum(c_lo)
            s_hi = plsc.cumsum(c_hi) + t_lo
            total = jnp.sum(c_hi) + t_lo
            off_v[pl.ds(0, L)] = s_lo - c_lo
            off_v[pl.ds(L, L)] = s_hi - c_hi
            cw_v[pl.ds(0, L)] = c_lo
            cw_v[pl.ds(L, L)] = c_hi
            pnum = jnp.minimum(total, K)
            for cp in copies:
                cp.wait()
            for jg in range(K // L):
                j = jg * L + iota
                w = jnp.zeros((L,), jnp.int32)
                for s in (16, 8, 4, 2, 1):
                    nxt = w + s
                    ow = plsc.load_gather(off_v, [nxt])
                    w = jnp.where(ow <= j, nxt, w)
                og = plsc.load_gather(off_v, [w])
                rel = jnp.minimum(jnp.maximum(j - og, 0), K - 1)
                src = plsc.load_gather(vbuf, [w * K + rel])
                out_v[pl.ds(jg * L, L)] = jnp.where(j < pnum, src, 0)
            pltpu.sync_copy(out_v, pidx_h.at[pl.ds(m * K, K)])
            plsc.store_scatter(
                pn_v, [jnp.full((L,), i, jnp.int32)],
                jnp.full((L,), pnum, jnp.int32), mask=iota == 0)
            return carry
        lax.fori_loop(0, boxes_per_w, box_body, 0)
        pltpu.sync_copy(pn_v.at[pl.ds(0, 8)], pnum_h.at[pl.ds(wid * 8, 8)])

    s2 = pl.kernel(
        stage2,
        out_type=[jax.ShapeDtypeStruct((nb * K,), jnp.int32),
                  jax.ShapeDtypeStruct((NW * 8,), jnp.int32)],
        mesh=_mesh(),
        compiler_params=pltpu.CompilerParams(needs_layout_passes=False),
        scratch_types=[
            pltpu.VMEM((NW * nb,), jnp.int32),
            pltpu.VMEM((NW * K,), jnp.int32),
            pltpu.VMEM((K,), jnp.int32),
            pltpu.VMEM((2 * L,), jnp.int32),
            pltpu.VMEM((2 * L,), jnp.int32),
            pltpu.VMEM((L,), jnp.int32),
            pltpu.SemaphoreType.DMA,
        ],
    )
    return s1, s2


def kernel(points, boxes3d):
    n = points.shape[0]
    b, m = boxes3d.shape[0], boxes3d.shape[1]
    nb = b * m
    groups = -(-n // (NW * L))
    n_pad = NW * groups * L
    s1, s2 = _build(n_pad, nb, groups, m, b)

    ctr = boxes3d[..., 0:3]
    edx = boxes3d[..., 4] + 2.0 * POOL_EXTRA
    edy = boxes3d[..., 3] + 2.0 * POOL_EXTRA
    edz = boxes3d[..., 5] + 2.0 * POOL_EXTRA
    ry = -boxes3d[..., 6] - np.float32(np.pi / 2.0)
    radius = 0.5 * jnp.sqrt(edx ** 2 + edy ** 2)
    par = jnp.stack([
        ctr[..., 0].reshape(nb), ctr[..., 1].reshape(nb),
        ctr[..., 2].reshape(nb),
        (edx * 0.5).reshape(nb), (edy * 0.5).reshape(nb),
        (edz * 0.5).reshape(nb),
        jnp.cos(ry).reshape(nb), jnp.sin(ry).reshape(nb),
        (radius + PS).reshape(nb),
    ], axis=1).reshape(9 * nb).astype(jnp.float32)

    pad = n_pad - n
    xs = jnp.concatenate([points[:, 1], jnp.zeros((pad,), jnp.float32)])
    ys = jnp.concatenate([points[:, 2], jnp.zeros((pad,), jnp.float32)])
    zs = jnp.concatenate([points[:, 3], jnp.zeros((pad,), jnp.float32)])
    pb = jnp.concatenate([points[:, 0], jnp.full((pad,), b, jnp.float32)])

    bufs, cnts = s1(xs, ys, zs, pb, par)
    pidx, pnum = s2(bufs, cnts)
    boxes_per_w = nb // NW
    pooled_idx = pidx.reshape(b, m, K)
    pooled_num = pnum.reshape(NW, 8)[:, :boxes_per_w].reshape(b, m)
    return pooled_idx, pooled_num
